# initial kernel scaffold (unmeasured)
import jax
import jax.numpy as jnp
from jax import lax
from jax.experimental import pallas as pl
from jax.experimental.pallas import tpu as pltpu

N_DEV = 8
M, K, N = 4096, 4096, 8192
CH = M // N_DEV


def _allreduce_body(
    p_ref,
    out_ref,
    recv_hbm,
    pvm,
    rvm,
    svm,
    rs_send_sems, rs_recv_sems, ag_send_sems, ag_recv_sems,
    csem,
):
    d = lax.axis_index("i")
    right = lax.rem(d + 1, N_DEV)

    for t in range(N_DEV - 1):
        c = lax.rem(d - 1 - t + 2 * N_DEV, N_DEV)
        cp = pltpu.make_async_copy(p_ref.at[pl.ds(c * CH, CH), :], pvm, csem)
        cp.start()
        cp.wait()
        if t == 0:
            svm[...] = pvm[...]
        else:
            cr = pltpu.make_async_copy(recv_hbm.at[t - 1], rvm, csem)
            cr.start()
            cr.wait()
            svm[...] = pvm[...] + rvm[...]
        rdma = pltpu.make_async_remote_copy(
            src_ref=svm,
            dst_ref=recv_hbm.at[t],
            send_sem=rs_send_sems.at[t],
            recv_sem=rs_recv_sems.at[t],
            device_id=(right,),
            device_id_type=pl.DeviceIdType.MESH,
        )
        rdma.start()
        rdma.wait()

    cr = pltpu.make_async_copy(recv_hbm.at[N_DEV - 2], rvm, csem)
    cr.start()
    cr.wait()
    cp = pltpu.make_async_copy(p_ref.at[pl.ds(d * CH, CH), :], pvm, csem)
    cp.start()
    cp.wait()
    svm[...] = pvm[...] + rvm[...]
    st = pltpu.make_async_copy(svm, out_ref.at[pl.ds(d * CH, CH), :], csem)
    st.start()
    st.wait()

    for t in range(N_DEV - 1):
        cs = lax.rem(d - t + 2 * N_DEV, N_DEV)
        rdma = pltpu.make_async_remote_copy(
            src_ref=out_ref.at[pl.ds(cs * CH, CH), :],
            dst_ref=out_ref.at[pl.ds(cs * CH, CH), :],
            send_sem=ag_send_sems.at[t],
            recv_sem=ag_recv_sems.at[t],
            device_id=(right,),
            device_id_type=pl.DeviceIdType.MESH,
        )
        rdma.start()
        rdma.wait()


def _ring_allreduce(p):
    return pl.pallas_call(
        _allreduce_body,
        out_shape=jax.ShapeDtypeStruct((M, N), jnp.bfloat16),
        in_specs=[pl.BlockSpec(memory_space=pl.ANY)],
        out_specs=pl.BlockSpec(memory_space=pl.ANY),
        scratch_shapes=[
            pltpu.MemorySpace.HBM((N_DEV - 1, CH, N), jnp.bfloat16),
            pltpu.MemorySpace.VMEM((CH, N), jnp.bfloat16),
            pltpu.MemorySpace.VMEM((CH, N), jnp.bfloat16),
            pltpu.MemorySpace.VMEM((CH, N), jnp.bfloat16),
            pltpu.SemaphoreType.DMA((N_DEV - 1,)),
            pltpu.SemaphoreType.DMA((N_DEV - 1,)),
            pltpu.SemaphoreType.DMA((N_DEV - 1,)),
            pltpu.SemaphoreType.DMA((N_DEV - 1,)),
            pltpu.SemaphoreType.DMA,
        ],
        compiler_params=pltpu.CompilerParams(has_side_effects=True),
    )(p)


def kernel(x, w_mat):
    p = jnp.dot(x, w_mat, preferred_element_type=jnp.float32)
    y = _ring_allreduce(p.astype(jnp.bfloat16)).astype(jnp.float32)
    amax = jnp.max(jnp.abs(y))
    scale = amax / 127.0
    q = jnp.clip(jnp.round(y / scale), -127.0, 127.0)
    return q * scale


# baseline (device time: 1502304 ns/iter reference)
import jax
import jax.numpy as jnp
from jax import lax
from jax.experimental import pallas as pl
from jax.experimental.pallas import tpu as pltpu

N_DEV = 8
M, K, N = 4096, 4096, 8192
CH = M // N_DEV


def _allreduce_body(
    p_ref,
    out_ref,
    recv_hbm,
    pvm,
    rvm,
    svm,
    rs_send_sems, rs_recv_sems, ag_send_sems, ag_recv_sems,
    csem,
):
    d = lax.axis_index("i")
    right = lax.rem(d + 1, N_DEV)

    for t in range(N_DEV - 1):
        c = lax.rem(d - 1 - t + 2 * N_DEV, N_DEV)
        cp = pltpu.make_async_copy(p_ref.at[pl.ds(c * CH, CH), :], pvm, csem)
        cp.start()
        cp.wait()
        if t == 0:
            svm[...] = pvm[...]
        else:
            cr = pltpu.make_async_copy(recv_hbm.at[t - 1], rvm, csem)
            cr.start()
            cr.wait()
            svm[...] = pvm[...] + rvm[...]
        rdma = pltpu.make_async_remote_copy(
            src_ref=svm,
            dst_ref=recv_hbm.at[t],
            send_sem=rs_send_sems.at[t],
            recv_sem=rs_recv_sems.at[t],
            device_id=(right,),
            device_id_type=pl.DeviceIdType.MESH,
        )
        rdma.start()
        rdma.wait()

    cr = pltpu.make_async_copy(recv_hbm.at[N_DEV - 2], rvm, csem)
    cr.start()
    cr.wait()
    cp = pltpu.make_async_copy(p_ref.at[pl.ds(d * CH, CH), :], pvm, csem)
    cp.start()
    cp.wait()
    svm[...] = pvm[...] + rvm[...]
    st = pltpu.make_async_copy(svm, out_ref.at[pl.ds(d * CH, CH), :], csem)
    st.start()
    st.wait()

    for t in range(N_DEV - 1):
        cs = lax.rem(d - t + 2 * N_DEV, N_DEV)
        rdma = pltpu.make_async_remote_copy(
            src_ref=out_ref.at[pl.ds(cs * CH, CH), :],
            dst_ref=out_ref.at[pl.ds(cs * CH, CH), :],
            send_sem=ag_send_sems.at[t],
            recv_sem=ag_recv_sems.at[t],
            device_id=(right,),
            device_id_type=pl.DeviceIdType.MESH,
        )
        rdma.start()
        rdma.wait()


def _ring_allreduce(p):
    y, _ = pl.pallas_call(
        _allreduce_body,
        out_shape=(
            jax.ShapeDtypeStruct((M, N), jnp.bfloat16),
            jax.ShapeDtypeStruct((N_DEV - 1, CH, N), jnp.bfloat16),
        ),
        in_specs=[pl.BlockSpec(memory_space=pl.ANY)],
        out_specs=(
            pl.BlockSpec(memory_space=pl.ANY),
            pl.BlockSpec(memory_space=pl.ANY),
        ),
        scratch_shapes=[
            pltpu.MemorySpace.VMEM((CH, N), jnp.bfloat16),
            pltpu.MemorySpace.VMEM((CH, N), jnp.bfloat16),
            pltpu.MemorySpace.VMEM((CH, N), jnp.bfloat16),
            pltpu.SemaphoreType.DMA((N_DEV - 1,)),
            pltpu.SemaphoreType.DMA((N_DEV - 1,)),
            pltpu.SemaphoreType.DMA((N_DEV - 1,)),
            pltpu.SemaphoreType.DMA((N_DEV - 1,)),
            pltpu.SemaphoreType.DMA,
        ],
        compiler_params=pltpu.CompilerParams(has_side_effects=True),
    )(p)
    return y


def kernel(x, w_mat):
    p = jnp.dot(x, w_mat, preferred_element_type=jnp.float32)
    y = _ring_allreduce(p.astype(jnp.bfloat16)).astype(jnp.float32)
    amax = jnp.max(jnp.abs(y))
    scale = amax / 127.0
    q = jnp.clip(jnp.round(y / scale), -127.0, 127.0)
    return q * scale


# device time: 894381 ns/iter; 1.6797x vs baseline; 1.6797x over previous
import jax
import jax.numpy as jnp
from jax import lax
from jax.experimental import pallas as pl
from jax.experimental.pallas import tpu as pltpu

N_DEV = 8
M, K, N = 4096, 4096, 8192
CH = M // N_DEV
N2 = N // 2


def _allreduce_body(
    p_ref,
    out_ref,
    recv_hbm,
    pvr, rvr, svr,
    pvl, rvl, svl,
    rs_send_r, rs_recv_r, ag_send_r, ag_recv_r,
    rs_send_l, rs_recv_l, ag_send_l, ag_recv_l,
    csem,
):
    d = lax.axis_index("i")
    right = lax.rem(d + 1, N_DEV)
    left = lax.rem(d - 1 + N_DEV, N_DEV)

    def _local(src, dst):
        cp = pltpu.make_async_copy(src, dst, csem)
        cp.start()
        cp.wait()

    for t in range(N_DEV - 1):
        cr = lax.rem(d - 1 - t + 2 * N_DEV, N_DEV)
        cl = lax.rem(d + 1 + t, N_DEV)
        _local(p_ref.at[pl.ds(cr * CH, CH), :N2], pvr)
        _local(p_ref.at[pl.ds(cl * CH, CH), N2:], pvl)
        if t == 0:
            svr[...] = pvr[...]
            svl[...] = pvl[...]
        else:
            _local(recv_hbm.at[t - 1, :, :N2], rvr)
            _local(recv_hbm.at[t - 1, :, N2:], rvl)
            svr[...] = pvr[...] + rvr[...]
            svl[...] = pvl[...] + rvl[...]
        rdma_r = pltpu.make_async_remote_copy(
            src_ref=svr,
            dst_ref=recv_hbm.at[t, :, :N2],
            send_sem=rs_send_r.at[t],
            recv_sem=rs_recv_r.at[t],
            device_id=(right,),
            device_id_type=pl.DeviceIdType.MESH,
        )
        rdma_l = pltpu.make_async_remote_copy(
            src_ref=svl,
            dst_ref=recv_hbm.at[t, :, N2:],
            send_sem=rs_send_l.at[t],
            recv_sem=rs_recv_l.at[t],
            device_id=(left,),
            device_id_type=pl.DeviceIdType.MESH,
        )
        rdma_r.start()
        rdma_l.start()
        rdma_r.wait()
        rdma_l.wait()

    _local(recv_hbm.at[N_DEV - 2, :, :N2], rvr)
    _local(recv_hbm.at[N_DEV - 2, :, N2:], rvl)
    _local(p_ref.at[pl.ds(d * CH, CH), :N2], pvr)
    _local(p_ref.at[pl.ds(d * CH, CH), N2:], pvl)
    svr[...] = pvr[...] + rvr[...]
    svl[...] = pvl[...] + rvl[...]
    _local(svr, out_ref.at[pl.ds(d * CH, CH), :N2])
    _local(svl, out_ref.at[pl.ds(d * CH, CH), N2:])

    for t in range(N_DEV - 1):
        cr = lax.rem(d - t + 2 * N_DEV, N_DEV)
        cl = lax.rem(d + t, N_DEV)
        rdma_r = pltpu.make_async_remote_copy(
            src_ref=out_ref.at[pl.ds(cr * CH, CH), :N2],
            dst_ref=out_ref.at[pl.ds(cr * CH, CH), :N2],
            send_sem=ag_send_r.at[t],
            recv_sem=ag_recv_r.at[t],
            device_id=(right,),
            device_id_type=pl.DeviceIdType.MESH,
        )
        rdma_l = pltpu.make_async_remote_copy(
            src_ref=out_ref.at[pl.ds(cl * CH, CH), N2:],
            dst_ref=out_ref.at[pl.ds(cl * CH, CH), N2:],
            send_sem=ag_send_l.at[t],
            recv_sem=ag_recv_l.at[t],
            device_id=(left,),
            device_id_type=pl.DeviceIdType.MESH,
        )
        rdma_r.start()
        rdma_l.start()
        rdma_r.wait()
        rdma_l.wait()


def _ring_allreduce(p):
    y, _ = pl.pallas_call(
        _allreduce_body,
        out_shape=(
            jax.ShapeDtypeStruct((M, N), jnp.bfloat16),
            jax.ShapeDtypeStruct((N_DEV - 1, CH, N), jnp.bfloat16),
        ),
        in_specs=[pl.BlockSpec(memory_space=pl.ANY)],
        out_specs=(
            pl.BlockSpec(memory_space=pl.ANY),
            pl.BlockSpec(memory_space=pl.ANY),
        ),
        scratch_shapes=[
            pltpu.MemorySpace.VMEM((CH, N2), jnp.bfloat16),
            pltpu.MemorySpace.VMEM((CH, N2), jnp.bfloat16),
            pltpu.MemorySpace.VMEM((CH, N2), jnp.bfloat16),
            pltpu.MemorySpace.VMEM((CH, N2), jnp.bfloat16),
            pltpu.MemorySpace.VMEM((CH, N2), jnp.bfloat16),
            pltpu.MemorySpace.VMEM((CH, N2), jnp.bfloat16),
            pltpu.SemaphoreType.DMA((N_DEV - 1,)),
            pltpu.SemaphoreType.DMA((N_DEV - 1,)),
            pltpu.SemaphoreType.DMA((N_DEV - 1,)),
            pltpu.SemaphoreType.DMA((N_DEV - 1,)),
            pltpu.SemaphoreType.DMA((N_DEV - 1,)),
            pltpu.SemaphoreType.DMA((N_DEV - 1,)),
            pltpu.SemaphoreType.DMA((N_DEV - 1,)),
            pltpu.SemaphoreType.DMA((N_DEV - 1,)),
            pltpu.SemaphoreType.DMA,
        ],
        compiler_params=pltpu.CompilerParams(has_side_effects=True),
    )(p)
    return y


def kernel(x, w_mat):
    p = jnp.dot(x, w_mat, preferred_element_type=jnp.float32)
    y = _ring_allreduce(p.astype(jnp.bfloat16)).astype(jnp.float32)
    amax = jnp.max(jnp.abs(y))
    scale = amax / 127.0
    q = jnp.clip(jnp.round(y / scale), -127.0, 127.0)
    return q * scale


# device time: 862028 ns/iter; 1.7428x vs baseline; 1.0375x over previous
import jax
import jax.numpy as jnp
from jax import lax
from jax.experimental import pallas as pl
from jax.experimental.pallas import tpu as pltpu

N_DEV = 8
M, K, N = 4096, 4096, 8192
CH = M // N_DEV
N2 = N // 2


def _allreduce_body(
    p_ref,
    out_ref,
    recv_hbm,
    pv, rvr, svr,
    rvl, svl,
    rs_send_r, rs_recv_r, ag_send_r, ag_recv_r,
    rs_send_l, rs_recv_l, ag_send_l, ag_recv_l,
    csem, pfsem,
):
    d = lax.axis_index("i")
    right = lax.rem(d + 1, N_DEV)
    left = lax.rem(d - 1 + N_DEV, N_DEV)

    def _local(src, dst):
        cp = pltpu.make_async_copy(src, dst, csem)
        cp.start()
        cp.wait()

    def _prefetch(t, slot):
        cr = lax.rem(d - 1 - t + 2 * N_DEV, N_DEV)
        cl = lax.rem(d + 1 + t, N_DEV)
        pltpu.make_async_copy(
            p_ref.at[pl.ds(cr * CH, CH), :N2], pv.at[slot, 0], pfsem.at[0]
        ).start()
        pltpu.make_async_copy(
            p_ref.at[pl.ds(cl * CH, CH), N2:], pv.at[slot, 1], pfsem.at[1]
        ).start()

    def _wait_prefetch(slot):
        pltpu.make_async_copy(p_ref.at[pl.ds(0, CH), :N2], pv.at[slot, 0], pfsem.at[0]).wait()
        pltpu.make_async_copy(p_ref.at[pl.ds(0, CH), N2:], pv.at[slot, 1], pfsem.at[1]).wait()

    _prefetch(0, 0)
    prev = None
    for t in range(N_DEV - 1):
        slot = t % 2
        _wait_prefetch(slot)
        if t == 0:
            svr[...] = pv[slot, 0]
            svl[...] = pv[slot, 1]
        else:
            prev[0].wait()
            prev[1].wait()
            _local(recv_hbm.at[t - 1, :, :N2], rvr)
            _local(recv_hbm.at[t - 1, :, N2:], rvl)
            svr[...] = pv[slot, 0] + rvr[...]
            svl[...] = pv[slot, 1] + rvl[...]
        rdma_r = pltpu.make_async_remote_copy(
            src_ref=svr,
            dst_ref=recv_hbm.at[t, :, :N2],
            send_sem=rs_send_r.at[t],
            recv_sem=rs_recv_r.at[t],
            device_id=(right,),
            device_id_type=pl.DeviceIdType.MESH,
        )
        rdma_l = pltpu.make_async_remote_copy(
            src_ref=svl,
            dst_ref=recv_hbm.at[t, :, N2:],
            send_sem=rs_send_l.at[t],
            recv_sem=rs_recv_l.at[t],
            device_id=(left,),
            device_id_type=pl.DeviceIdType.MESH,
        )
        rdma_r.start()
        rdma_l.start()
        _prefetch(t + 1, (t + 1) % 2)
        prev = (rdma_r, rdma_l)

    prev[0].wait()
    prev[1].wait()
    slot = (N_DEV - 1) % 2
    _wait_prefetch(slot)
    _local(recv_hbm.at[N_DEV - 2, :, :N2], rvr)
    _local(recv_hbm.at[N_DEV - 2, :, N2:], rvl)
    svr[...] = pv[slot, 0] + rvr[...]
    svl[...] = pv[slot, 1] + rvl[...]
    _local(svr, out_ref.at[pl.ds(d * CH, CH), :N2])
    _local(svl, out_ref.at[pl.ds(d * CH, CH), N2:])

    for t in range(N_DEV - 1):
        cr = lax.rem(d - t + 2 * N_DEV, N_DEV)
        cl = lax.rem(d + t, N_DEV)
        rdma_r = pltpu.make_async_remote_copy(
            src_ref=out_ref.at[pl.ds(cr * CH, CH), :N2],
            dst_ref=out_ref.at[pl.ds(cr * CH, CH), :N2],
            send_sem=ag_send_r.at[t],
            recv_sem=ag_recv_r.at[t],
            device_id=(right,),
            device_id_type=pl.DeviceIdType.MESH,
        )
        rdma_l = pltpu.make_async_remote_copy(
            src_ref=out_ref.at[pl.ds(cl * CH, CH), N2:],
            dst_ref=out_ref.at[pl.ds(cl * CH, CH), N2:],
            send_sem=ag_send_l.at[t],
            recv_sem=ag_recv_l.at[t],
            device_id=(left,),
            device_id_type=pl.DeviceIdType.MESH,
        )
        rdma_r.start()
        rdma_l.start()
        rdma_r.wait()
        rdma_l.wait()


def _ring_allreduce(p):
    y, _ = pl.pallas_call(
        _allreduce_body,
        out_shape=(
            jax.ShapeDtypeStruct((M, N), jnp.bfloat16),
            jax.ShapeDtypeStruct((N_DEV - 1, CH, N), jnp.bfloat16),
        ),
        in_specs=[pl.BlockSpec(memory_space=pl.ANY)],
        out_specs=(
            pl.BlockSpec(memory_space=pl.ANY),
            pl.BlockSpec(memory_space=pl.ANY),
        ),
        scratch_shapes=[
            pltpu.MemorySpace.VMEM((2, 2, CH, N2), jnp.bfloat16),
            pltpu.MemorySpace.VMEM((CH, N2), jnp.bfloat16),
            pltpu.MemorySpace.VMEM((CH, N2), jnp.bfloat16),
            pltpu.MemorySpace.VMEM((CH, N2), jnp.bfloat16),
            pltpu.MemorySpace.VMEM((CH, N2), jnp.bfloat16),
            pltpu.SemaphoreType.DMA((N_DEV - 1,)),
            pltpu.SemaphoreType.DMA((N_DEV - 1,)),
            pltpu.SemaphoreType.DMA((N_DEV - 1,)),
            pltpu.SemaphoreType.DMA((N_DEV - 1,)),
            pltpu.SemaphoreType.DMA((N_DEV - 1,)),
            pltpu.SemaphoreType.DMA((N_DEV - 1,)),
            pltpu.SemaphoreType.DMA((N_DEV - 1,)),
            pltpu.SemaphoreType.DMA((N_DEV - 1,)),
            pltpu.SemaphoreType.DMA,
            pltpu.SemaphoreType.DMA((2,)),
        ],
        compiler_params=pltpu.CompilerParams(has_side_effects=True),
    )(p)
    return y


def kernel(x, w_mat):
    p = jnp.dot(x, w_mat, preferred_element_type=jnp.bfloat16)
    y = _ring_allreduce(p).astype(jnp.float32)
    amax = jnp.max(jnp.abs(y))
    scale = amax / 127.0
    q = jnp.clip(jnp.round(y / scale), -127.0, 127.0)
    return q * scale
